# baseline (speedup 1.0000x reference)
import functools

import jax
import jax.numpy as jnp
from jax import lax
from jax.experimental import pallas as pl
from jax.experimental.pallas import tpu as pltpu
from jax.experimental.pallas import tpu_sc as plsc

N_NODES = 10000
N_EDGES = 160000
IN_FEAT = 128
OUT_FEAT = 128
NUM_RELS = 8
NUM_RULES = 4
D = NUM_RULES * OUT_FEAT

NC = 2
NS = 16
L = 16

NW = NC * NS
S = 160
NRANGE = (N_NODES + S - 1) // S
NROUND = (NRANGE + NW - 1) // NW
CH = 1600
NCHT = N_EDGES // CH
MW = 6
B = 32
LCAP = 2048


def _proj_body(f_ref, w_ref, o_ref):
    f = f_ref[...]
    w = w_ref[0]
    for r in range(NUM_RULES):
        sl = slice(r * OUT_FEAT, (r + 1) * OUT_FEAT)
        o_ref[:, sl] = jnp.dot(f[:, sl], w, preferred_element_type=jnp.float32)


def _project(feat2d, weight):
    nb = 10
    bn = N_NODES // nb
    return pl.pallas_call(
        _proj_body,
        grid=(nb, NUM_RELS),
        in_specs=[
            pl.BlockSpec((bn, D), lambda i, k: (i, 0)),
            pl.BlockSpec((1, IN_FEAT, OUT_FEAT), lambda i, k: (k, 0, 0)),
        ],
        out_specs=pl.BlockSpec((bn, D), lambda i, k: (k * nb + i, 0)),
        out_shape=jax.ShapeDtypeStruct((NUM_RELS * N_NODES, D), jnp.float32),
    )(feat2d, weight)


@functools.cache
def _make_sc_scatter():
    mesh = plsc.VectorSubcoreMesh(
        core_axis_name="c", subcore_axis_name="s", num_cores=NC, num_subcores=NS
    )
    return pl.kernel(
        _sc_scatter_body,
        out_type=jax.ShapeDtypeStruct((N_NODES * D,), jnp.float32),
        mesh=mesh,
        scratch_types=[
        pltpu.VMEM((CH * MW,), jnp.int32),
        pltpu.VMEM((CH * MW,), jnp.int32),
        pltpu.VMEM((LCAP,), jnp.int32),
        pltpu.VMEM((LCAP + L,), jnp.int32),
        pltpu.VMEM((NUM_RULES * LCAP,), jnp.float32),
        pltpu.VMEM((B, D), jnp.float32),
        pltpu.VMEM((S * D,), jnp.float32),
        pltpu.SemaphoreType.DMA,
        pltpu.SemaphoreType.DMA,
        pltpu.SemaphoreType.DMA,
        ],
        compiler_params=pltpu.CompilerParams(needs_layout_passes=False),
    )


def _sc_scatter_body(proj_hbm, meta_hbm, zeros_hbm, out_hbm,
                     mbuf0, mbuf1, mgrow, mdloc, mcd, rows, acc,
                     sem0, sem1, semg):
    cid = lax.axis_index("c")
    sid = lax.axis_index("s")
    wid = sid * NC + cid

    def process_batch(b):
        gi = mgrow.at[pl.ds(b * B, B)]
        pltpu.async_copy(proj_hbm.at[gi], rows, semg).wait()

        def edge_body(i, _):
            dlv = mdloc[pl.ds(b * B + i, L)]
            base = dlv[0] * D
            ei = jnp.full((L,), b * B + i, jnp.int32)
            for r in range(NUM_RULES):
                c = plsc.load_gather(mcd, [ei + r * LCAP])
                for v in range(OUT_FEAT // L):
                    off = r * OUT_FEAT + v * L
                    plsc.addupdate(acc.at[pl.ds(base + off, L)],
                                   rows[i, pl.ds(off, L)] * c)
            return 0

        lax.fori_loop(0, B, edge_body, 0, unroll=2)

    for rnd in range(NROUND):
        rid = rnd * NW + wid

        @pl.when(rid < NRANGE)
        def _round():
            lo = rid * S
            pltpu.sync_copy(zeros_hbm, acc)

            def scan_chunk(mbuf, nv):
                def scan_body(j, nv):
                    idx6 = (jnp.arange(L, dtype=jnp.int32) * MW +
                            j * (L * MW))
                    d = plsc.load_gather(mbuf, [idx6 + 1])
                    dl = d - lo
                    m = (dl >= 0) & (dl < S)
                    mi = jnp.where(m, jnp.ones((L,), jnp.int32),
                                   jnp.zeros((L,), jnp.int32))
                    g = plsc.load_gather(mbuf, [idx6])
                    pos = nv + plsc.cumsum(mi) - 1
                    plsc.store_scatter(mgrow, [pos], g, mask=m)
                    plsc.store_scatter(mdloc, [pos], dl, mask=m)
                    for r in range(NUM_RULES):
                        c = plsc.bitcast(
                            plsc.load_gather(mbuf, [idx6 + (2 + r)]),
                            jnp.float32)
                        plsc.store_scatter(mcd, [pos + r * LCAP], c,
                                           mask=m)
                    return nv + plsc.all_reduce_population_count(m)

                nv = plsc.parallel_loop(
                    0, CH // L, step=1, unroll=4, carry=nv)(scan_body)
                n = nv[0]
                nb = n // B

                def batch_body(b, _):
                    process_batch(b)
                    return 0

                lax.fori_loop(0, nb, batch_body, 0)
                rem_base = nb * B
                for t in range(B // L):
                    sl_src = pl.ds(rem_base + t * L, L)
                    sl_dst = pl.ds(t * L, L)
                    mgrow[sl_dst] = mgrow[sl_src]
                    mdloc[sl_dst] = mdloc[sl_src]
                    for r in range(NUM_RULES):
                        mcd[pl.ds(r * LCAP + t * L, L)] = mcd[
                            pl.ds(r * LCAP + rem_base + t * L, L)]
                return nv - rem_base

            def meta_at(ch):
                return meta_hbm.at[pl.ds(ch * (CH * MW), CH * MW)]

            pltpu.async_copy(meta_at(0), mbuf0, sem0)

            def pair_body(i, nv):
                ch0 = 2 * i
                pltpu.make_async_copy(meta_at(ch0), mbuf0, sem0).wait()
                pltpu.async_copy(meta_at(ch0 + 1), mbuf1, sem1)
                nv = scan_chunk(mbuf0, nv)
                pltpu.make_async_copy(meta_at(ch0 + 1), mbuf1, sem1).wait()
                nxt = jnp.minimum(ch0 + 2, NCHT - 2)
                pltpu.async_copy(meta_at(nxt), mbuf0, sem0)
                nv = scan_chunk(mbuf1, nv)
                return nv

            nv = lax.fori_loop(0, NCHT // 2, pair_body,
                               jnp.zeros((L,), jnp.int32))
            n = nv[0]
            pltpu.make_async_copy(meta_at(NCHT - 2), mbuf0, sem0).wait()

            for t in range(B // L):
                sl = pl.ds(n + t * L, L)
                mgrow[sl] = jnp.zeros((L,), jnp.int32)
                mdloc[sl] = jnp.zeros((L,), jnp.int32)
                for r in range(NUM_RULES):
                    mcd[pl.ds(r * LCAP + n + t * L, L)] = jnp.zeros(
                        (L,), jnp.float32)

            @pl.when(n > 0)
            def _():
                process_batch(0)

            @pl.when(rid < NRANGE - 1)
            def _():
                pltpu.sync_copy(acc, out_hbm.at[pl.ds(rid * (S * D), S * D)])

            @pl.when(rid == NRANGE - 1)
            def _():
                rem = (N_NODES - (NRANGE - 1) * S) * D
                pltpu.sync_copy(acc.at[pl.ds(0, rem)],
                                out_hbm.at[pl.ds(rid * (S * D), rem)])


def kernel(feat, edge_index, etypes, coupling_degree, weight, h_bias):
    del h_bias
    src = edge_index[0].astype(jnp.int32)
    dst = edge_index[1].astype(jnp.int32)
    et = etypes.astype(jnp.int32)

    feat2d = feat.reshape(N_NODES, D)
    proj = _project(feat2d, weight)

    grow = et * N_NODES + src
    zeros = jnp.zeros((S * D,), jnp.float32)

    cdbits = jax.lax.bitcast_convert_type(coupling_degree, jnp.int32)
    meta = jnp.concatenate(
        [grow[:, None], dst[:, None], cdbits], axis=1
    ).reshape(N_EDGES * MW)
    out2d = _make_sc_scatter()(proj, meta, zeros)
    return out2d.reshape(N_NODES, NUM_RULES, OUT_FEAT)


# speedup vs baseline: 1.0124x; 1.0124x over previous
import functools

import jax
import jax.numpy as jnp
from jax import lax
from jax.experimental import pallas as pl
from jax.experimental.pallas import tpu as pltpu
from jax.experimental.pallas import tpu_sc as plsc

N_NODES = 10000
N_EDGES = 160000
IN_FEAT = 128
OUT_FEAT = 128
NUM_RELS = 8
NUM_RULES = 4
D = NUM_RULES * OUT_FEAT

NC = 2
NS = 16
L = 16

NW = NC * NS
S = 160
NRANGE = (N_NODES + S - 1) // S
NROUND = (NRANGE + NW - 1) // NW
CH = 1600
NCHT = N_EDGES // CH
MW = 6
B = 32
LCAP = 2048


def _proj_body(f_ref, w_ref, o_ref):
    f = f_ref[...]
    w = w_ref[0]
    for r in range(NUM_RULES):
        sl = slice(r * OUT_FEAT, (r + 1) * OUT_FEAT)
        o_ref[:, sl] = jnp.dot(f[:, sl], w, preferred_element_type=jnp.float32)


def _project(feat2d, weight):
    nb = 10
    bn = N_NODES // nb
    return pl.pallas_call(
        _proj_body,
        grid=(nb, NUM_RELS),
        in_specs=[
            pl.BlockSpec((bn, D), lambda i, k: (i, 0)),
            pl.BlockSpec((1, IN_FEAT, OUT_FEAT), lambda i, k: (k, 0, 0)),
        ],
        out_specs=pl.BlockSpec((bn, D), lambda i, k: (k * nb + i, 0)),
        out_shape=jax.ShapeDtypeStruct((NUM_RELS * N_NODES, D), jnp.float32),
    )(feat2d, weight)


@functools.cache
def _make_sc_scatter():
    mesh = plsc.VectorSubcoreMesh(
        core_axis_name="c", subcore_axis_name="s", num_cores=NC, num_subcores=NS
    )
    return pl.kernel(
        _sc_scatter_body,
        out_type=jax.ShapeDtypeStruct((N_NODES * D,), jnp.float32),
        mesh=mesh,
        scratch_types=[
        pltpu.VMEM((CH * MW,), jnp.int32),
        pltpu.VMEM((CH * MW,), jnp.int32),
        pltpu.VMEM((LCAP,), jnp.int32),
        pltpu.VMEM((LCAP + L,), jnp.int32),
        pltpu.VMEM((NUM_RULES * LCAP,), jnp.float32),
        pltpu.VMEM((B, D), jnp.float32),
        pltpu.VMEM((S * D,), jnp.float32),
        pltpu.SemaphoreType.DMA,
        pltpu.SemaphoreType.DMA,
        pltpu.SemaphoreType.DMA,
        ],
        compiler_params=pltpu.CompilerParams(needs_layout_passes=False),
    )


def _sc_scatter_body(proj_hbm, meta_hbm, zeros_hbm, out_hbm,
                     mbuf0, mbuf1, mgrow, mdloc, mcd, rows, acc,
                     sem0, sem1, semg):
    cid = lax.axis_index("c")
    sid = lax.axis_index("s")
    wid = sid * NC + cid

    def process_batch(b):
        gi = mgrow.at[pl.ds(b * B, B)]
        pltpu.async_copy(proj_hbm.at[gi], rows, semg).wait()

        def edge_body(i, _):
            ei = jnp.full((L,), b * B + i, jnp.int32)
            rowbase = plsc.load_gather(mdloc, [ei]) * D
            for r in range(NUM_RULES):
                c = plsc.load_gather(mcd, [ei + r * LCAP])
                for v in range(OUT_FEAT // L):
                    off = r * OUT_FEAT + v * L
                    idx = rowbase + (jnp.arange(L, dtype=jnp.int32) + off)
                    plsc.addupdate_scatter(
                        acc, [idx], rows[i, pl.ds(off, L)] * c)
            return 0

        lax.fori_loop(0, B, edge_body, 0, unroll=4)

    for rnd in range(NROUND):
        rid = rnd * NW + wid

        @pl.when(rid < NRANGE)
        def _round():
            lo = rid * S
            pltpu.sync_copy(zeros_hbm, acc)

            def scan_chunk(mbuf, nv):
                def scan_body(j, nv):
                    idx6 = (jnp.arange(L, dtype=jnp.int32) * MW +
                            j * (L * MW))
                    d = plsc.load_gather(mbuf, [idx6 + 1])
                    dl = d - lo
                    m = (dl >= 0) & (dl < S)
                    mi = jnp.where(m, jnp.ones((L,), jnp.int32),
                                   jnp.zeros((L,), jnp.int32))
                    g = plsc.load_gather(mbuf, [idx6])
                    pos = nv + plsc.cumsum(mi) - 1
                    plsc.store_scatter(mgrow, [pos], g, mask=m)
                    plsc.store_scatter(mdloc, [pos], dl, mask=m)
                    for r in range(NUM_RULES):
                        c = plsc.bitcast(
                            plsc.load_gather(mbuf, [idx6 + (2 + r)]),
                            jnp.float32)
                        plsc.store_scatter(mcd, [pos + r * LCAP], c,
                                           mask=m)
                    return nv + plsc.all_reduce_population_count(m)

                nv = plsc.parallel_loop(
                    0, CH // L, step=1, unroll=4, carry=nv)(scan_body)
                n = nv[0]
                nb = n // B

                def batch_body(b, _):
                    process_batch(b)
                    return 0

                lax.fori_loop(0, nb, batch_body, 0)
                rem_base = nb * B
                for t in range(B // L):
                    sl_src = pl.ds(rem_base + t * L, L)
                    sl_dst = pl.ds(t * L, L)
                    mgrow[sl_dst] = mgrow[sl_src]
                    mdloc[sl_dst] = mdloc[sl_src]
                    for r in range(NUM_RULES):
                        mcd[pl.ds(r * LCAP + t * L, L)] = mcd[
                            pl.ds(r * LCAP + rem_base + t * L, L)]
                return nv - rem_base

            def meta_at(ch):
                return meta_hbm.at[pl.ds(ch * (CH * MW), CH * MW)]

            pltpu.async_copy(meta_at(0), mbuf0, sem0)

            def pair_body(i, nv):
                ch0 = 2 * i
                pltpu.make_async_copy(meta_at(ch0), mbuf0, sem0).wait()
                pltpu.async_copy(meta_at(ch0 + 1), mbuf1, sem1)
                nv = scan_chunk(mbuf0, nv)
                pltpu.make_async_copy(meta_at(ch0 + 1), mbuf1, sem1).wait()
                nxt = jnp.minimum(ch0 + 2, NCHT - 2)
                pltpu.async_copy(meta_at(nxt), mbuf0, sem0)
                nv = scan_chunk(mbuf1, nv)
                return nv

            nv = lax.fori_loop(0, NCHT // 2, pair_body,
                               jnp.zeros((L,), jnp.int32))
            n = nv[0]
            pltpu.make_async_copy(meta_at(NCHT - 2), mbuf0, sem0).wait()

            for t in range(B // L):
                sl = pl.ds(n + t * L, L)
                mgrow[sl] = jnp.zeros((L,), jnp.int32)
                mdloc[sl] = jnp.zeros((L,), jnp.int32)
                for r in range(NUM_RULES):
                    mcd[pl.ds(r * LCAP + n + t * L, L)] = jnp.zeros(
                        (L,), jnp.float32)

            @pl.when(n > 0)
            def _():
                process_batch(0)

            @pl.when(rid < NRANGE - 1)
            def _():
                pltpu.sync_copy(acc, out_hbm.at[pl.ds(rid * (S * D), S * D)])

            @pl.when(rid == NRANGE - 1)
            def _():
                rem = (N_NODES - (NRANGE - 1) * S) * D
                pltpu.sync_copy(acc.at[pl.ds(0, rem)],
                                out_hbm.at[pl.ds(rid * (S * D), rem)])


def kernel(feat, edge_index, etypes, coupling_degree, weight, h_bias):
    del h_bias
    src = edge_index[0].astype(jnp.int32)
    dst = edge_index[1].astype(jnp.int32)
    et = etypes.astype(jnp.int32)

    feat2d = feat.reshape(N_NODES, D)
    proj = _project(feat2d, weight)

    grow = et * N_NODES + src
    zeros = jnp.zeros((S * D,), jnp.float32)

    cdbits = jax.lax.bitcast_convert_type(coupling_degree, jnp.int32)
    meta = jnp.concatenate(
        [grow[:, None], dst[:, None], cdbits], axis=1
    ).reshape(N_EDGES * MW)
    out2d = _make_sc_scatter()(proj, meta, zeros)
    return out2d.reshape(N_NODES, NUM_RULES, OUT_FEAT)


# speedup vs baseline: 1.0157x; 1.0033x over previous
import functools

import jax
import jax.numpy as jnp
from jax import lax
from jax.experimental import pallas as pl
from jax.experimental.pallas import tpu as pltpu
from jax.experimental.pallas import tpu_sc as plsc

N_NODES = 10000
N_EDGES = 160000
IN_FEAT = 128
OUT_FEAT = 128
NUM_RELS = 8
NUM_RULES = 4
D = NUM_RULES * OUT_FEAT

NC = 2
NS = 16
L = 16

NW = NC * NS
S = 160
NRANGE = (N_NODES + S - 1) // S
NROUND = (NRANGE + NW - 1) // NW
CH = 1600
NCHT = N_EDGES // CH
MW = 6
B = 32
LCAP = 2048


def _proj_body(f_ref, w_ref, o_ref):
    f = f_ref[...]
    w = w_ref[0]
    for r in range(NUM_RULES):
        sl = slice(r * OUT_FEAT, (r + 1) * OUT_FEAT)
        o_ref[:, sl] = jnp.dot(f[:, sl], w, preferred_element_type=jnp.float32)


def _project(feat2d, weight):
    nb = 10
    bn = N_NODES // nb
    return pl.pallas_call(
        _proj_body,
        grid=(nb, NUM_RELS),
        in_specs=[
            pl.BlockSpec((bn, D), lambda i, k: (i, 0)),
            pl.BlockSpec((1, IN_FEAT, OUT_FEAT), lambda i, k: (k, 0, 0)),
        ],
        out_specs=pl.BlockSpec((bn, D), lambda i, k: (k * nb + i, 0)),
        out_shape=jax.ShapeDtypeStruct((NUM_RELS * N_NODES, D), jnp.float32),
    )(feat2d, weight)


@functools.cache
def _make_sc_scatter():
    mesh = plsc.VectorSubcoreMesh(
        core_axis_name="c", subcore_axis_name="s", num_cores=NC, num_subcores=NS
    )
    return pl.kernel(
        _sc_scatter_body,
        out_type=jax.ShapeDtypeStruct((N_NODES * D,), jnp.float32),
        mesh=mesh,
        scratch_types=[
        pltpu.VMEM((CH * MW,), jnp.int32),
        pltpu.VMEM((CH * MW,), jnp.int32),
        pltpu.VMEM((LCAP,), jnp.int32),
        pltpu.VMEM((LCAP + L,), jnp.int32),
        pltpu.VMEM((NUM_RULES * LCAP,), jnp.float32),
        pltpu.VMEM((B, D), jnp.float32),
        pltpu.VMEM((S * D,), jnp.float32),
        pltpu.SemaphoreType.DMA,
        pltpu.SemaphoreType.DMA,
        pltpu.SemaphoreType.DMA,
        ],
        compiler_params=pltpu.CompilerParams(needs_layout_passes=False),
    )


def _sc_scatter_body(proj_hbm, meta_hbm, zeros_hbm, out_hbm,
                     mbuf0, mbuf1, mgrow, mdloc, mcd, rows, acc,
                     sem0, sem1, semg):
    cid = lax.axis_index("c")
    sid = lax.axis_index("s")
    wid = sid * NC + cid

    def process_batch(b):
        gi = mgrow.at[pl.ds(b * B, B)]
        pltpu.async_copy(proj_hbm.at[gi], rows, semg).wait()

        def edge_body(i, _):
            ei = jnp.full((L,), b * B + i, jnp.int32)
            rowbase = plsc.load_gather(mdloc, [ei]) * D
            for r in range(NUM_RULES):
                c = plsc.load_gather(mcd, [ei + r * LCAP])
                for v in range(OUT_FEAT // L):
                    off = r * OUT_FEAT + v * L
                    idx = rowbase + (jnp.arange(L, dtype=jnp.int32) + off)
                    plsc.addupdate_scatter(
                        acc, [idx], rows[i, pl.ds(off, L)] * c)
            return 0

        lax.fori_loop(0, B, edge_body, 0, unroll=2)

    for rnd in range(NROUND):
        rid = rnd * NW + wid

        @pl.when(rid < NRANGE)
        def _round():
            lo = rid * S
            pltpu.sync_copy(zeros_hbm, acc)

            def scan_chunk(mbuf, nv):
                def scan_body(j, nv):
                    idx6 = (jnp.arange(L, dtype=jnp.int32) * MW +
                            j * (L * MW))
                    d = plsc.load_gather(mbuf, [idx6 + 1])
                    dl = d - lo
                    m = (dl >= 0) & (dl < S)
                    mi = jnp.where(m, jnp.ones((L,), jnp.int32),
                                   jnp.zeros((L,), jnp.int32))
                    g = plsc.load_gather(mbuf, [idx6])
                    pos = nv + plsc.cumsum(mi) - 1
                    plsc.store_scatter(mgrow, [pos], g, mask=m)
                    plsc.store_scatter(mdloc, [pos], dl, mask=m)
                    for r in range(NUM_RULES):
                        c = plsc.bitcast(
                            plsc.load_gather(mbuf, [idx6 + (2 + r)]),
                            jnp.float32)
                        plsc.store_scatter(mcd, [pos + r * LCAP], c,
                                           mask=m)
                    return nv + plsc.all_reduce_population_count(m)

                nv = plsc.parallel_loop(
                    0, CH // L, step=1, unroll=4, carry=nv)(scan_body)
                n = nv[0]
                nb = n // B

                def batch_body(b, _):
                    process_batch(b)
                    return 0

                lax.fori_loop(0, nb, batch_body, 0)
                rem_base = nb * B
                for t in range(B // L):
                    sl_src = pl.ds(rem_base + t * L, L)
                    sl_dst = pl.ds(t * L, L)
                    mgrow[sl_dst] = mgrow[sl_src]
                    mdloc[sl_dst] = mdloc[sl_src]
                    for r in range(NUM_RULES):
                        mcd[pl.ds(r * LCAP + t * L, L)] = mcd[
                            pl.ds(r * LCAP + rem_base + t * L, L)]
                return nv - rem_base

            def meta_at(ch):
                return meta_hbm.at[pl.ds(ch * (CH * MW), CH * MW)]

            pltpu.async_copy(meta_at(0), mbuf0, sem0)

            def pair_body(i, nv):
                ch0 = 2 * i
                pltpu.make_async_copy(meta_at(ch0), mbuf0, sem0).wait()
                pltpu.async_copy(meta_at(ch0 + 1), mbuf1, sem1)
                nv = scan_chunk(mbuf0, nv)
                pltpu.make_async_copy(meta_at(ch0 + 1), mbuf1, sem1).wait()
                nxt = jnp.minimum(ch0 + 2, NCHT - 2)
                pltpu.async_copy(meta_at(nxt), mbuf0, sem0)
                nv = scan_chunk(mbuf1, nv)
                return nv

            nv = lax.fori_loop(0, NCHT // 2, pair_body,
                               jnp.zeros((L,), jnp.int32))
            n = nv[0]
            pltpu.make_async_copy(meta_at(NCHT - 2), mbuf0, sem0).wait()

            for t in range(B // L):
                sl = pl.ds(n + t * L, L)
                mgrow[sl] = jnp.zeros((L,), jnp.int32)
                mdloc[sl] = jnp.zeros((L,), jnp.int32)
                for r in range(NUM_RULES):
                    mcd[pl.ds(r * LCAP + n + t * L, L)] = jnp.zeros(
                        (L,), jnp.float32)

            @pl.when(n > 0)
            def _():
                process_batch(0)

            @pl.when(rid < NRANGE - 1)
            def _():
                pltpu.sync_copy(acc, out_hbm.at[pl.ds(rid * (S * D), S * D)])

            @pl.when(rid == NRANGE - 1)
            def _():
                rem = (N_NODES - (NRANGE - 1) * S) * D
                pltpu.sync_copy(acc.at[pl.ds(0, rem)],
                                out_hbm.at[pl.ds(rid * (S * D), rem)])


def kernel(feat, edge_index, etypes, coupling_degree, weight, h_bias):
    del h_bias
    src = edge_index[0].astype(jnp.int32)
    dst = edge_index[1].astype(jnp.int32)
    et = etypes.astype(jnp.int32)

    feat2d = feat.reshape(N_NODES, D)
    proj = _project(feat2d, weight)

    grow = et * N_NODES + src
    zeros = jnp.zeros((S * D,), jnp.float32)

    cdbits = jax.lax.bitcast_convert_type(coupling_degree, jnp.int32)
    meta = jnp.concatenate(
        [grow[:, None], dst[:, None], cdbits], axis=1
    ).reshape(N_EDGES * MW)
    out2d = _make_sc_scatter()(proj, meta, zeros)
    return out2d.reshape(N_NODES, NUM_RULES, OUT_FEAT)
